# baseline (speedup 1.0000x reference)
import functools

import jax
import jax.numpy as jnp
from jax import lax
from jax.experimental import pallas as pl
from jax.experimental.pallas import tpu as pltpu
from jax.experimental.pallas import tpu_sc as plsc

N = 10000
E = 160000
DIM = 256
HALF = 128

NC = 2
NS = 16
CHUNK = 80
NCHUNKS = E // CHUNK
CPS = NCHUNKS // NS
SLOTS = 4
WIN = 25
NA = N
RPS = N // NS

_sc_mesh = plsc.VectorSubcoreMesh(core_axis_name="c", subcore_axis_name="s")


@functools.partial(
    pl.kernel,
    out_type=jax.ShapeDtypeStruct((2 * N, HALF), jnp.float32),
    mesh=_sc_mesh,
    scratch_types=[
        pltpu.VMEM((2 * WIN, CHUNK), jnp.int32),
        pltpu.VMEM((2 * WIN, CHUNK), jnp.int32),
        pltpu.VMEM((SLOTS, CHUNK, HALF), jnp.float32),
        pltpu.VMEM_SHARED((NA, HALF), jnp.float32),
        pltpu.SemaphoreType.DMA((SLOTS,)),
        pltpu.SemaphoreType.DMA((SLOTS,)),
        pltpu.SemaphoreType.DMA((2,)),
        pltpu.SemaphoreType.DMA,
    ],
    compiler_params=pltpu.CompilerParams(use_tc_tiling_on_sc=False),
)
def _sc_aggregate(pk_hbm, h_hbm, zer_hbm, out_hbm,
                  sidx_v, didx_v, rows_v, accum,
                  sem_g, sem_s, sem_k, sem_i):
    c = lax.axis_index("c")
    s = lax.axis_index("s")

    zpart = RPS // 5

    def _zissue(t, carry):
        pltpu.async_copy(zer_hbm.at[pl.ds(0, zpart)],
                         accum.at[pl.ds(s * RPS + t * zpart, zpart)], sem_i)
        return carry

    lax.fori_loop(0, 5, _zissue, 0)

    pltpu.async_copy(pk_hbm.at[c, pl.ds(s * CPS, WIN)],
                     sidx_v.at[pl.ds(0, WIN)], sem_k.at[0])
    pltpu.async_copy(pk_hbm.at[c, pl.ds(s * CPS + WIN, WIN)],
                     sidx_v.at[pl.ds(WIN, WIN)], sem_k.at[1])

    def _zwait(t, carry):
        pltpu.make_async_copy(zer_hbm.at[pl.ds(0, zpart)],
                              accum.at[pl.ds(s * RPS + t * zpart, zpart)],
                              sem_i).wait()
        return carry

    lax.fori_loop(0, 5, _zwait, 0)

    plsc.subcore_barrier()

    def _step(j, carry):
        slot = j % SLOTS
        pslot = (j + SLOTS - 1) % SLOTS
        w = j // WIN

        @pl.when(jnp.logical_and(j % WIN == 0, j < CPS))
        def _win():
            @pl.when(jnp.logical_and(w >= 1, w + 1 < CPS // WIN))
            def _issue_win():
                ww = w + 1
                pltpu.async_copy(pk_hbm.at[c, pl.ds(s * CPS + ww * WIN, WIN)],
                                 sidx_v.at[pl.ds((ww % 2) * WIN, WIN)],
                                 sem_k.at[ww % 2])
            pltpu.make_async_copy(pk_hbm.at[c, pl.ds(s * CPS + w * WIN, WIN)],
                                  sidx_v.at[pl.ds((w % 2) * WIN, WIN)],
                                  sem_k.at[w % 2]).wait()

            def _unp(t, carry2):
                r = (w % 2) * WIN + t
                for u in range(CHUNK // 16):
                    v = sidx_v[r, pl.ds(u * 16, 16)]
                    didx_v[r, pl.ds(u * 16, 16)] = (
                        lax.shift_right_logical(v, 15))
                    sidx_v[r, pl.ds(u * 16, 16)] = v & 0x7FFF
                return carry2

            lax.fori_loop(0, WIN, _unp, 0)

        @pl.when(jnp.logical_and(j >= SLOTS, j - SLOTS < CPS))
        def _wait_scatter():
            jj = j - SLOTS
            rr = (jj // WIN % 2) * WIN + jj % WIN
            pltpu.make_async_copy(rows_v.at[slot], accum.at[didx_v.at[rr]],
                                  sem_s.at[slot]).wait()

        @pl.when(j < CPS)
        def _issue():
            r = (w % 2) * WIN + j % WIN
            pltpu.async_copy(h_hbm.at[sidx_v.at[r]], rows_v.at[slot],
                             sem_g.at[slot])

        @pl.when(jnp.logical_and(j > 0, j <= CPS))
        def _drain():
            jd = j - 1
            rd = (jd // WIN % 2) * WIN + jd % WIN
            pltpu.make_async_copy(h_hbm.at[sidx_v.at[rd]], rows_v.at[pslot],
                                  sem_g.at[pslot]).wait()
            pltpu.async_copy(rows_v.at[pslot], accum.at[didx_v.at[rd]],
                             sem_s.at[pslot], add=True)
        return carry

    lax.fori_loop(0, CPS + SLOTS, _step, 0)

    plsc.subcore_barrier()

    def _wissue(t, carry):
        pltpu.async_copy(accum.at[pl.ds(s * RPS + t * zpart, zpart)],
                         out_hbm.at[pl.ds(c * N + s * RPS + t * zpart, zpart)],
                         sem_i)
        return carry

    lax.fori_loop(0, 5, _wissue, 0)

    def _wwait(t, carry):
        pltpu.make_async_copy(accum.at[pl.ds(s * RPS + t * zpart, zpart)],
                              out_hbm.at[pl.ds(c * N + s * RPS + t * zpart,
                                               zpart)], sem_i).wait()
        return carry

    lax.fori_loop(0, 5, _wwait, 0)


BLK = 1000
NBLK = N // BLK


def _dense_body(hlo_ref, hhi_ref, alo_ref, ahi_ref, w1_ref, b1_ref,
                w2_ref, b2_ref, z_ref, stats_ref):
    hlo = hlo_ref[...] + alo_ref[...]
    hhi = hhi_ref[...] + ahi_ref[...]
    w1 = w1_ref[...]
    y = jnp.dot(hlo, w1[:HALF, :], preferred_element_type=jnp.float32)
    y += jnp.dot(hhi, w1[HALF:, :], preferred_element_type=jnp.float32)
    y = jnp.maximum(y + b1_ref[...], 0.0)
    z = jnp.dot(y, w2_ref[...], preferred_element_type=jnp.float32)
    z = jnp.maximum(z + b2_ref[...], 0.0)
    z_ref[...] = z
    part = jnp.concatenate(
        [jnp.sum(z, axis=0, keepdims=True),
         jnp.sum(z * z, axis=0, keepdims=True),
         jnp.zeros((6, DIM), jnp.float32)], axis=0)

    @pl.when(pl.program_id(0) == 0)
    def _():
        stats_ref[...] = jnp.zeros_like(stats_ref)

    stats_ref[...] += part


_dense = pl.pallas_call(
    _dense_body,
    grid=(NBLK,),
    in_specs=[
        pl.BlockSpec((BLK, HALF), lambda j: (j, 0)),
        pl.BlockSpec((BLK, HALF), lambda j: (NBLK + j, 0)),
        pl.BlockSpec((BLK, HALF), lambda j: (j, 0)),
        pl.BlockSpec((BLK, HALF), lambda j: (NBLK + j, 0)),
        pl.BlockSpec((DIM, DIM), lambda j: (0, 0)),
        pl.BlockSpec((1, DIM), lambda j: (0, 0)),
        pl.BlockSpec((DIM, DIM), lambda j: (0, 0)),
        pl.BlockSpec((1, DIM), lambda j: (0, 0)),
    ],
    out_specs=[
        pl.BlockSpec((BLK, DIM), lambda j: (j, 0)),
        pl.BlockSpec((8, DIM), lambda j: (0, 0)),
    ],
    out_shape=[
        jax.ShapeDtypeStruct((N, DIM), jnp.float32),
        jax.ShapeDtypeStruct((8, DIM), jnp.float32),
    ],
)


def _norm_split_body(z_ref, stats_ref, g_ref, be_ref, out_ref):
    mean = stats_ref[0:1, :] / N
    var = stats_ref[1:2, :] / N - mean * mean
    inv = g_ref[...] * lax.rsqrt(var + 1e-5)
    out_ref[...] = (z_ref[...] - mean) * inv + be_ref[...]


_norm_split = pl.pallas_call(
    _norm_split_body,
    grid=(2, NBLK),
    in_specs=[
        pl.BlockSpec((BLK, HALF), lambda i, j: (j, i)),
        pl.BlockSpec((8, HALF), lambda i, j: (0, i)),
        pl.BlockSpec((1, HALF), lambda i, j: (0, i)),
        pl.BlockSpec((1, HALF), lambda i, j: (0, i)),
    ],
    out_specs=pl.BlockSpec((BLK, HALF), lambda i, j: (i * NBLK + j, 0)),
    out_shape=jax.ShapeDtypeStruct((2 * N, HALF), jnp.float32),
)


def _norm_full_body(z_ref, stats_ref, g_ref, be_ref, out_ref):
    mean = stats_ref[0:1, :] / N
    var = stats_ref[1:2, :] / N - mean * mean
    inv = g_ref[...] * lax.rsqrt(var + 1e-5)
    out_ref[...] = (z_ref[...] - mean) * inv + be_ref[...]


_norm_full = pl.pallas_call(
    _norm_full_body,
    grid=(NBLK,),
    in_specs=[
        pl.BlockSpec((BLK, DIM), lambda j: (j, 0)),
        pl.BlockSpec((8, DIM), lambda j: (0, 0)),
        pl.BlockSpec((1, DIM), lambda j: (0, 0)),
        pl.BlockSpec((1, DIM), lambda j: (0, 0)),
    ],
    out_specs=pl.BlockSpec((BLK, DIM), lambda j: (j, 0)),
    out_shape=jax.ShapeDtypeStruct((N, DIM), jnp.float32),
)


def kernel(x, edge_index, w11, b11, w12, b12, g1, be1,
           w21, b21, w22, b22, g2, be2,
           w31, b31, w32, b32, g3, be3):
    src = edge_index[0].astype(jnp.int32)
    dst = edge_index[1].astype(jnp.int32)
    dsh = dst << 15
    pk = jnp.stack([src | dsh, (src + N) | dsh]).reshape(2, NCHUNKS, CHUNK)

    h = jnp.concatenate([x[:, :HALF], x[:, HALF:]], axis=0)

    layers = [(w11, b11, w12, b12, g1, be1),
              (w21, b21, w22, b22, g2, be2),
              (w31, b31, w32, b32, g3, be3)]

    zer = jnp.zeros((RPS, HALF), jnp.float32)
    for li, (w1, b1, w2, b2, g, be) in enumerate(layers):
        agg = _sc_aggregate(pk, h, zer)
        z, stats = _dense(h, h, agg, agg, w1, b1.reshape(1, DIM),
                          w2, b2.reshape(1, DIM))
        if li < 2:
            h = _norm_split(z, stats, g.reshape(1, DIM), be.reshape(1, DIM))
        else:
            return _norm_full(z, stats, g.reshape(1, DIM), be.reshape(1, DIM))


# speedup vs baseline: 1.0790x; 1.0790x over previous
import functools

import jax
import jax.numpy as jnp
from jax import lax
from jax.experimental import pallas as pl
from jax.experimental.pallas import tpu as pltpu
from jax.experimental.pallas import tpu_sc as plsc

N = 10000
E = 160000
DIM = 256
HALF = 128

NC = 2
NS = 16
CHUNK = 80
NCHUNKS = E // CHUNK
CPS = NCHUNKS // NS
SLOTS = 3
WIN = 25
NA = N
RPS = N // NS

_sc_mesh = plsc.VectorSubcoreMesh(core_axis_name="c", subcore_axis_name="s")


@functools.partial(
    pl.kernel,
    out_type=jax.ShapeDtypeStruct((2 * N, HALF), jnp.float32),
    mesh=_sc_mesh,
    scratch_types=[
        pltpu.VMEM((2 * WIN, CHUNK), jnp.int32),
        pltpu.VMEM((2 * WIN, CHUNK), jnp.int32),
        pltpu.VMEM((SLOTS, CHUNK, HALF), jnp.float32),
        pltpu.VMEM_SHARED((NA, HALF), jnp.float32),
        pltpu.SemaphoreType.DMA((SLOTS,)),
        pltpu.SemaphoreType.DMA((SLOTS,)),
        pltpu.SemaphoreType.DMA((2,)),
        pltpu.SemaphoreType.DMA,
    ],
    compiler_params=pltpu.CompilerParams(use_tc_tiling_on_sc=False),
)
def _sc_aggregate(pk_hbm, h_hbm, zer_hbm, out_hbm,
                  sidx_v, didx_v, rows_v, accum,
                  sem_g, sem_s, sem_k, sem_i):
    c = lax.axis_index("c")
    s = lax.axis_index("s")

    zpart = RPS // 5

    def _zissue(t, carry):
        pltpu.async_copy(zer_hbm.at[pl.ds(0, zpart)],
                         accum.at[pl.ds(s * RPS + t * zpart, zpart)], sem_i)
        return carry

    lax.fori_loop(0, 5, _zissue, 0)

    pltpu.async_copy(pk_hbm.at[c, pl.ds(s * CPS, WIN)],
                     sidx_v.at[pl.ds(0, WIN)], sem_k.at[0])
    pltpu.async_copy(pk_hbm.at[c, pl.ds(s * CPS + WIN, WIN)],
                     sidx_v.at[pl.ds(WIN, WIN)], sem_k.at[1])

    def _zwait(t, carry):
        pltpu.make_async_copy(zer_hbm.at[pl.ds(0, zpart)],
                              accum.at[pl.ds(s * RPS + t * zpart, zpart)],
                              sem_i).wait()
        return carry

    lax.fori_loop(0, 5, _zwait, 0)

    plsc.subcore_barrier()

    def _step(j, carry):
        slot = j % SLOTS
        pslot = (j + SLOTS - 1) % SLOTS
        w = j // WIN

        @pl.when(jnp.logical_and(j % WIN == 0, j < CPS))
        def _win():
            @pl.when(jnp.logical_and(w >= 1, w + 1 < CPS // WIN))
            def _issue_win():
                ww = w + 1
                pltpu.async_copy(pk_hbm.at[c, pl.ds(s * CPS + ww * WIN, WIN)],
                                 sidx_v.at[pl.ds((ww % 2) * WIN, WIN)],
                                 sem_k.at[ww % 2])
            pltpu.make_async_copy(pk_hbm.at[c, pl.ds(s * CPS + w * WIN, WIN)],
                                  sidx_v.at[pl.ds((w % 2) * WIN, WIN)],
                                  sem_k.at[w % 2]).wait()

            def _unp(t, carry2):
                r = (w % 2) * WIN + t
                for u in range(CHUNK // 16):
                    v = sidx_v[r, pl.ds(u * 16, 16)]
                    didx_v[r, pl.ds(u * 16, 16)] = (
                        lax.shift_right_logical(v, 15))
                    sidx_v[r, pl.ds(u * 16, 16)] = v & 0x7FFF
                return carry2

            lax.fori_loop(0, WIN, _unp, 0)

        @pl.when(jnp.logical_and(j >= SLOTS, j - SLOTS < CPS))
        def _wait_scatter():
            jj = j - SLOTS
            rr = (jj // WIN % 2) * WIN + jj % WIN
            pltpu.make_async_copy(rows_v.at[slot], accum.at[didx_v.at[rr]],
                                  sem_s.at[slot]).wait()

        @pl.when(j < CPS)
        def _issue():
            r = (w % 2) * WIN + j % WIN
            pltpu.async_copy(h_hbm.at[sidx_v.at[r]], rows_v.at[slot],
                             sem_g.at[slot])

        @pl.when(jnp.logical_and(j > 0, j <= CPS))
        def _drain():
            jd = j - 1
            rd = (jd // WIN % 2) * WIN + jd % WIN
            pltpu.make_async_copy(h_hbm.at[sidx_v.at[rd]], rows_v.at[pslot],
                                  sem_g.at[pslot]).wait()
            pltpu.async_copy(rows_v.at[pslot], accum.at[didx_v.at[rd]],
                             sem_s.at[pslot], add=True)
        return carry

    lax.fori_loop(0, CPS + SLOTS, _step, 0)

    plsc.subcore_barrier()

    def _wissue(t, carry):
        pltpu.async_copy(accum.at[pl.ds(s * RPS + t * zpart, zpart)],
                         out_hbm.at[pl.ds(c * N + s * RPS + t * zpart, zpart)],
                         sem_i)
        return carry

    lax.fori_loop(0, 5, _wissue, 0)

    def _wwait(t, carry):
        pltpu.make_async_copy(accum.at[pl.ds(s * RPS + t * zpart, zpart)],
                              out_hbm.at[pl.ds(c * N + s * RPS + t * zpart,
                                               zpart)], sem_i).wait()
        return carry

    lax.fori_loop(0, 5, _wwait, 0)


BLK = 1000
NBLK = N // BLK


def _dense_body(hlo_ref, hhi_ref, alo_ref, ahi_ref, w1_ref, b1_ref,
                w2_ref, b2_ref, z_ref, stats_ref):
    hlo = hlo_ref[...] + alo_ref[...]
    hhi = hhi_ref[...] + ahi_ref[...]
    w1 = w1_ref[...]
    y = jnp.dot(hlo, w1[:HALF, :], preferred_element_type=jnp.float32)
    y += jnp.dot(hhi, w1[HALF:, :], preferred_element_type=jnp.float32)
    y = jnp.maximum(y + b1_ref[...], 0.0)
    z = jnp.dot(y, w2_ref[...], preferred_element_type=jnp.float32)
    z = jnp.maximum(z + b2_ref[...], 0.0)
    z_ref[...] = z
    part = jnp.concatenate(
        [jnp.sum(z, axis=0, keepdims=True),
         jnp.sum(z * z, axis=0, keepdims=True),
         jnp.zeros((6, DIM), jnp.float32)], axis=0)

    @pl.when(pl.program_id(0) == 0)
    def _():
        stats_ref[...] = jnp.zeros_like(stats_ref)

    stats_ref[...] += part


_dense = pl.pallas_call(
    _dense_body,
    grid=(NBLK,),
    in_specs=[
        pl.BlockSpec((BLK, HALF), lambda j: (j, 0)),
        pl.BlockSpec((BLK, HALF), lambda j: (NBLK + j, 0)),
        pl.BlockSpec((BLK, HALF), lambda j: (j, 0)),
        pl.BlockSpec((BLK, HALF), lambda j: (NBLK + j, 0)),
        pl.BlockSpec((DIM, DIM), lambda j: (0, 0)),
        pl.BlockSpec((1, DIM), lambda j: (0, 0)),
        pl.BlockSpec((DIM, DIM), lambda j: (0, 0)),
        pl.BlockSpec((1, DIM), lambda j: (0, 0)),
    ],
    out_specs=[
        pl.BlockSpec((BLK, DIM), lambda j: (j, 0)),
        pl.BlockSpec((8, DIM), lambda j: (0, 0)),
    ],
    out_shape=[
        jax.ShapeDtypeStruct((N, DIM), jnp.float32),
        jax.ShapeDtypeStruct((8, DIM), jnp.float32),
    ],
)


def _norm_split_body(z_ref, stats_ref, g_ref, be_ref, out_ref):
    mean = stats_ref[0:1, :] / N
    var = stats_ref[1:2, :] / N - mean * mean
    inv = g_ref[...] * lax.rsqrt(var + 1e-5)
    out_ref[...] = (z_ref[...] - mean) * inv + be_ref[...]


_norm_split = pl.pallas_call(
    _norm_split_body,
    grid=(2, NBLK),
    in_specs=[
        pl.BlockSpec((BLK, HALF), lambda i, j: (j, i)),
        pl.BlockSpec((8, HALF), lambda i, j: (0, i)),
        pl.BlockSpec((1, HALF), lambda i, j: (0, i)),
        pl.BlockSpec((1, HALF), lambda i, j: (0, i)),
    ],
    out_specs=pl.BlockSpec((BLK, HALF), lambda i, j: (i * NBLK + j, 0)),
    out_shape=jax.ShapeDtypeStruct((2 * N, HALF), jnp.float32),
)


def _norm_full_body(z_ref, stats_ref, g_ref, be_ref, out_ref):
    mean = stats_ref[0:1, :] / N
    var = stats_ref[1:2, :] / N - mean * mean
    inv = g_ref[...] * lax.rsqrt(var + 1e-5)
    out_ref[...] = (z_ref[...] - mean) * inv + be_ref[...]


_norm_full = pl.pallas_call(
    _norm_full_body,
    grid=(NBLK,),
    in_specs=[
        pl.BlockSpec((BLK, DIM), lambda j: (j, 0)),
        pl.BlockSpec((8, DIM), lambda j: (0, 0)),
        pl.BlockSpec((1, DIM), lambda j: (0, 0)),
        pl.BlockSpec((1, DIM), lambda j: (0, 0)),
    ],
    out_specs=pl.BlockSpec((BLK, DIM), lambda j: (j, 0)),
    out_shape=jax.ShapeDtypeStruct((N, DIM), jnp.float32),
)


def kernel(x, edge_index, w11, b11, w12, b12, g1, be1,
           w21, b21, w22, b22, g2, be2,
           w31, b31, w32, b32, g3, be3):
    src = edge_index[0].astype(jnp.int32)
    dst = edge_index[1].astype(jnp.int32)
    dsh = dst << 15
    pk = jnp.stack([src | dsh, (src + N) | dsh]).reshape(2, NCHUNKS, CHUNK)

    h = jnp.concatenate([x[:, :HALF], x[:, HALF:]], axis=0)

    layers = [(w11, b11, w12, b12, g1, be1),
              (w21, b21, w22, b22, g2, be2),
              (w31, b31, w32, b32, g3, be3)]

    zer = jnp.zeros((RPS, HALF), jnp.float32)
    for li, (w1, b1, w2, b2, g, be) in enumerate(layers):
        agg = _sc_aggregate(pk, h, zer)
        z, stats = _dense(h, h, agg, agg, w1, b1.reshape(1, DIM),
                          w2, b2.reshape(1, DIM))
        if li < 2:
            h = _norm_split(z, stats, g.reshape(1, DIM), be.reshape(1, DIM))
        else:
            return _norm_full(z, stats, g.reshape(1, DIM), be.reshape(1, DIM))


# speedup vs baseline: 1.1945x; 1.1070x over previous
import functools

import jax
import jax.numpy as jnp
from jax import lax
from jax.experimental import pallas as pl
from jax.experimental.pallas import tpu as pltpu
from jax.experimental.pallas import tpu_sc as plsc

N = 10000
E = 160000
DIM = 256
HALF = 128

NC = 2
NS = 16
CHUNK = 80
PADE = E
NCHUNKS = PADE // CHUNK
CPS = NCHUNKS // NS
IDX_PARTS = 5
NA = N
RPS = N // NS
STAGE_ROWS = 25

_sc_mesh = plsc.VectorSubcoreMesh(core_axis_name="c", subcore_axis_name="s")


@functools.partial(
    pl.kernel,
    out_type=jax.ShapeDtypeStruct((2 * N, HALF), jnp.float32),
    mesh=_sc_mesh,
    scratch_types=[
        pltpu.VMEM((CPS, CHUNK), jnp.int32),
        pltpu.VMEM((CPS, CHUNK), jnp.int32),
        pltpu.VMEM((3, CHUNK, HALF), jnp.float32),
        pltpu.VMEM_SHARED((NA, HALF), jnp.float32),
        pltpu.SemaphoreType.DMA((3,)),
        pltpu.SemaphoreType.DMA((3,)),
        pltpu.SemaphoreType.DMA,
    ],
    compiler_params=pltpu.CompilerParams(use_tc_tiling_on_sc=False),
)
def _sc_aggregate(src2_hbm, dst_hbm, h_hbm, zer_hbm, out_hbm,
                  src_v, dst_v, rows_v, accum, sem_g, sem_s, sem_i):
    c = lax.axis_index("c")
    s = lax.axis_index("s")

    zcp = pltpu.async_copy(zer_hbm, accum.at[pl.ds(s * RPS, RPS)], sem_i)

    part = CPS // IDX_PARTS

    def _ldidx(t, carry):
        pltpu.async_copy(src2_hbm.at[c, pl.ds(s * CPS + t * part, part)],
                         src_v.at[pl.ds(t * part, part)], sem_g.at[0])
        pltpu.async_copy(dst_hbm.at[pl.ds(s * CPS + t * part, part)],
                         dst_v.at[pl.ds(t * part, part)], sem_g.at[1])
        return carry

    lax.fori_loop(0, IDX_PARTS, _ldidx, 0)

    def _ldwait(t, carry):
        pltpu.make_async_copy(src2_hbm.at[c, pl.ds(s * CPS + t * part, part)],
                              src_v.at[pl.ds(t * part, part)],
                              sem_g.at[0]).wait()
        pltpu.make_async_copy(dst_hbm.at[pl.ds(s * CPS + t * part, part)],
                              dst_v.at[pl.ds(t * part, part)],
                              sem_g.at[1]).wait()
        return carry

    lax.fori_loop(0, IDX_PARTS, _ldwait, 0)
    zcp.wait()

    plsc.subcore_barrier()

    def _step(j, carry):
        slot = j % 3
        pslot = (j + 2) % 3

        @pl.when(jnp.logical_and(j >= 3, j - 3 < CPS))
        def _wait_scatter():
            pltpu.make_async_copy(rows_v.at[slot],
                                  accum.at[dst_v.at[j - 3]],
                                  sem_s.at[slot]).wait()

        @pl.when(j < CPS)
        def _issue():
            pltpu.async_copy(h_hbm.at[src_v.at[j]], rows_v.at[slot],
                             sem_g.at[slot])

        @pl.when(jnp.logical_and(j > 0, j <= CPS))
        def _drain():
            pltpu.make_async_copy(h_hbm.at[src_v.at[j - 1]],
                                  rows_v.at[pslot], sem_g.at[pslot]).wait()
            pltpu.async_copy(rows_v.at[pslot], accum.at[dst_v.at[j - 1]],
                             sem_s.at[pslot], add=True)
        return carry

    lax.fori_loop(0, CPS + 3, _step, 0)

    plsc.subcore_barrier()

    pltpu.sync_copy(accum.at[pl.ds(s * RPS, RPS)],
                    out_hbm.at[pl.ds(c * N + s * RPS, RPS)])


BLK = 2000
NBLK = N // BLK


def _dense_body(hlo_ref, hhi_ref, alo_ref, ahi_ref, w1_ref, b1_ref,
                w2_ref, b2_ref, z_ref, stats_ref):
    hlo = hlo_ref[...] + alo_ref[...]
    hhi = hhi_ref[...] + ahi_ref[...]
    w1 = w1_ref[...]
    y = jnp.dot(hlo, w1[:HALF, :], preferred_element_type=jnp.float32)
    y += jnp.dot(hhi, w1[HALF:, :], preferred_element_type=jnp.float32)
    y = jnp.maximum(y + b1_ref[...], 0.0)
    z = jnp.dot(y, w2_ref[...], preferred_element_type=jnp.float32)
    z = jnp.maximum(z + b2_ref[...], 0.0)
    z_ref[...] = z
    part = jnp.concatenate(
        [jnp.sum(z, axis=0, keepdims=True),
         jnp.sum(z * z, axis=0, keepdims=True),
         jnp.zeros((6, DIM), jnp.float32)], axis=0)

    @pl.when(pl.program_id(0) == 0)
    def _():
        stats_ref[...] = jnp.zeros_like(stats_ref)

    stats_ref[...] += part


_dense = pl.pallas_call(
    _dense_body,
    grid=(NBLK,),
    in_specs=[
        pl.BlockSpec((BLK, HALF), lambda j: (j, 0)),
        pl.BlockSpec((BLK, HALF), lambda j: (NBLK + j, 0)),
        pl.BlockSpec((BLK, HALF), lambda j: (j, 0)),
        pl.BlockSpec((BLK, HALF), lambda j: (NBLK + j, 0)),
        pl.BlockSpec((DIM, DIM), lambda j: (0, 0)),
        pl.BlockSpec((1, DIM), lambda j: (0, 0)),
        pl.BlockSpec((DIM, DIM), lambda j: (0, 0)),
        pl.BlockSpec((1, DIM), lambda j: (0, 0)),
    ],
    out_specs=[
        pl.BlockSpec((BLK, DIM), lambda j: (j, 0)),
        pl.BlockSpec((8, DIM), lambda j: (0, 0)),
    ],
    out_shape=[
        jax.ShapeDtypeStruct((N, DIM), jnp.float32),
        jax.ShapeDtypeStruct((8, DIM), jnp.float32),
    ],
)


def _norm_split_body(z_ref, stats_ref, g_ref, be_ref, out_ref):
    mean = stats_ref[0:1, :] / N
    var = stats_ref[1:2, :] / N - mean * mean
    inv = g_ref[...] * lax.rsqrt(var + 1e-5)
    out_ref[...] = (z_ref[...] - mean) * inv + be_ref[...]


_norm_split = pl.pallas_call(
    _norm_split_body,
    grid=(2, NBLK),
    in_specs=[
        pl.BlockSpec((BLK, HALF), lambda i, j: (j, i)),
        pl.BlockSpec((8, HALF), lambda i, j: (0, i)),
        pl.BlockSpec((1, HALF), lambda i, j: (0, i)),
        pl.BlockSpec((1, HALF), lambda i, j: (0, i)),
    ],
    out_specs=pl.BlockSpec((BLK, HALF), lambda i, j: (i * NBLK + j, 0)),
    out_shape=jax.ShapeDtypeStruct((2 * N, HALF), jnp.float32),
)


def _norm_full_body(z_ref, stats_ref, g_ref, be_ref, out_ref):
    mean = stats_ref[0:1, :] / N
    var = stats_ref[1:2, :] / N - mean * mean
    inv = g_ref[...] * lax.rsqrt(var + 1e-5)
    out_ref[...] = (z_ref[...] - mean) * inv + be_ref[...]


_norm_full = pl.pallas_call(
    _norm_full_body,
    grid=(NBLK,),
    in_specs=[
        pl.BlockSpec((BLK, DIM), lambda j: (j, 0)),
        pl.BlockSpec((8, DIM), lambda j: (0, 0)),
        pl.BlockSpec((1, DIM), lambda j: (0, 0)),
        pl.BlockSpec((1, DIM), lambda j: (0, 0)),
    ],
    out_specs=pl.BlockSpec((BLK, DIM), lambda j: (j, 0)),
    out_shape=jax.ShapeDtypeStruct((N, DIM), jnp.float32),
)


def kernel(x, edge_index, w11, b11, w12, b12, g1, be1,
           w21, b21, w22, b22, g2, be2,
           w31, b31, w32, b32, g3, be3):
    src = edge_index[0].astype(jnp.int32)
    dst = edge_index[1].astype(jnp.int32)
    pad = PADE - E
    src = jnp.concatenate([src, jnp.zeros((pad,), jnp.int32)])
    dst = jnp.concatenate([dst, jnp.full((pad,), N, jnp.int32)])
    src2 = jnp.stack([src, src + N]).reshape(2, NCHUNKS, CHUNK)
    dstc = dst.reshape(NCHUNKS, CHUNK)

    h = jnp.concatenate([x[:, :HALF], x[:, HALF:]], axis=0)

    layers = [(w11, b11, w12, b12, g1, be1),
              (w21, b21, w22, b22, g2, be2),
              (w31, b31, w32, b32, g3, be3)]

    zer = jnp.zeros((RPS, HALF), jnp.float32)
    for li, (w1, b1, w2, b2, g, be) in enumerate(layers):
        agg = _sc_aggregate(src2, dstc, h, zer)
        z, stats = _dense(h, h, agg, agg, w1, b1.reshape(1, DIM),
                          w2, b2.reshape(1, DIM))
        if li < 2:
            h = _norm_split(z, stats, g.reshape(1, DIM), be.reshape(1, DIM))
        else:
            return _norm_full(z, stats, g.reshape(1, DIM), be.reshape(1, DIM))


# speedup vs baseline: 1.2362x; 1.0349x over previous
import functools

import jax
import jax.numpy as jnp
from jax import lax
from jax.experimental import pallas as pl
from jax.experimental.pallas import tpu as pltpu
from jax.experimental.pallas import tpu_sc as plsc

N = 10000
E = 160000
DIM = 256
HALF = 128

NC = 2
NS = 16
CHUNK = 80
PADE = E
NCHUNKS = PADE // CHUNK
CPS = NCHUNKS // NS
IDX_PARTS = 5
NA = N
RPS = N // NS
STAGE_ROWS = 25

_sc_mesh = plsc.VectorSubcoreMesh(core_axis_name="c", subcore_axis_name="s")


@functools.partial(
    pl.kernel,
    out_type=jax.ShapeDtypeStruct((2 * N, HALF), jnp.float32),
    mesh=_sc_mesh,
    scratch_types=[
        pltpu.VMEM((CPS, CHUNK), jnp.int32),
        pltpu.VMEM((CPS, CHUNK), jnp.int32),
        pltpu.VMEM((3, CHUNK, HALF), jnp.float32),
        pltpu.VMEM_SHARED((NA, HALF), jnp.float32),
        pltpu.SemaphoreType.DMA((3,)),
        pltpu.SemaphoreType.DMA((3,)),
        pltpu.SemaphoreType.DMA,
    ],
    compiler_params=pltpu.CompilerParams(use_tc_tiling_on_sc=False),
)
def _sc_aggregate(src2_hbm, dst_hbm, h_hbm, zer_hbm, out_hbm,
                  src_v, dst_v, rows_v, accum, sem_g, sem_s, sem_i):
    c = lax.axis_index("c")
    s = lax.axis_index("s")

    zcp = pltpu.async_copy(zer_hbm, accum.at[pl.ds(s * RPS, RPS)], sem_i)

    part = CPS // IDX_PARTS

    def _ldidx(t, carry):
        pltpu.async_copy(src2_hbm.at[c, pl.ds(s * CPS + t * part, part)],
                         src_v.at[pl.ds(t * part, part)], sem_g.at[0])
        pltpu.async_copy(dst_hbm.at[pl.ds(s * CPS + t * part, part)],
                         dst_v.at[pl.ds(t * part, part)], sem_g.at[1])
        return carry

    lax.fori_loop(0, IDX_PARTS, _ldidx, 0)

    def _ldwait(t, carry):
        pltpu.make_async_copy(src2_hbm.at[c, pl.ds(s * CPS + t * part, part)],
                              src_v.at[pl.ds(t * part, part)],
                              sem_g.at[0]).wait()
        pltpu.make_async_copy(dst_hbm.at[pl.ds(s * CPS + t * part, part)],
                              dst_v.at[pl.ds(t * part, part)],
                              sem_g.at[1]).wait()
        return carry

    lax.fori_loop(0, IDX_PARTS, _ldwait, 0)
    zcp.wait()

    plsc.subcore_barrier()

    def _step(j, carry):
        slot = j % 3
        pslot = (j + 2) % 3

        @pl.when(jnp.logical_and(j >= 3, j - 3 < CPS))
        def _wait_scatter():
            pltpu.make_async_copy(rows_v.at[slot],
                                  accum.at[dst_v.at[j - 3]],
                                  sem_s.at[slot]).wait()

        @pl.when(j < CPS)
        def _issue():
            pltpu.async_copy(h_hbm.at[src_v.at[j]], rows_v.at[slot],
                             sem_g.at[slot])

        @pl.when(jnp.logical_and(j > 0, j <= CPS))
        def _drain():
            pltpu.make_async_copy(h_hbm.at[src_v.at[j - 1]],
                                  rows_v.at[pslot], sem_g.at[pslot]).wait()
            pltpu.async_copy(rows_v.at[pslot], accum.at[dst_v.at[j - 1]],
                             sem_s.at[pslot], add=True)
        return carry

    lax.fori_loop(0, CPS + 3, _step, 0)

    plsc.subcore_barrier()

    pltpu.sync_copy(accum.at[pl.ds(s * RPS, RPS)],
                    out_hbm.at[pl.ds(c * N + s * RPS, RPS)])


BLK = 5000
NBLK = N // BLK


def _dense_body(hlo_ref, hhi_ref, alo_ref, ahi_ref, w1_ref, b1_ref,
                w2_ref, b2_ref, z_ref, stats_ref):
    hlo = hlo_ref[...] + alo_ref[...]
    hhi = hhi_ref[...] + ahi_ref[...]
    w1 = w1_ref[...]
    y = jnp.dot(hlo, w1[:HALF, :], preferred_element_type=jnp.float32)
    y += jnp.dot(hhi, w1[HALF:, :], preferred_element_type=jnp.float32)
    y = jnp.maximum(y + b1_ref[...], 0.0)
    z = jnp.dot(y, w2_ref[...], preferred_element_type=jnp.float32)
    z = jnp.maximum(z + b2_ref[...], 0.0)
    z_ref[...] = z
    part = jnp.concatenate(
        [jnp.sum(z, axis=0, keepdims=True),
         jnp.sum(z * z, axis=0, keepdims=True),
         jnp.zeros((6, DIM), jnp.float32)], axis=0)

    @pl.when(pl.program_id(0) == 0)
    def _():
        stats_ref[...] = jnp.zeros_like(stats_ref)

    stats_ref[...] += part


_dense = pl.pallas_call(
    _dense_body,
    grid=(NBLK,),
    in_specs=[
        pl.BlockSpec((BLK, HALF), lambda j: (j, 0)),
        pl.BlockSpec((BLK, HALF), lambda j: (NBLK + j, 0)),
        pl.BlockSpec((BLK, HALF), lambda j: (j, 0)),
        pl.BlockSpec((BLK, HALF), lambda j: (NBLK + j, 0)),
        pl.BlockSpec((DIM, DIM), lambda j: (0, 0)),
        pl.BlockSpec((1, DIM), lambda j: (0, 0)),
        pl.BlockSpec((DIM, DIM), lambda j: (0, 0)),
        pl.BlockSpec((1, DIM), lambda j: (0, 0)),
    ],
    out_specs=[
        pl.BlockSpec((BLK, DIM), lambda j: (j, 0)),
        pl.BlockSpec((8, DIM), lambda j: (0, 0)),
    ],
    out_shape=[
        jax.ShapeDtypeStruct((N, DIM), jnp.float32),
        jax.ShapeDtypeStruct((8, DIM), jnp.float32),
    ],
)


def _norm_split_body(z_ref, stats_ref, g_ref, be_ref, out_ref):
    mean = stats_ref[0:1, :] / N
    var = stats_ref[1:2, :] / N - mean * mean
    inv = g_ref[...] * lax.rsqrt(var + 1e-5)
    out_ref[...] = (z_ref[...] - mean) * inv + be_ref[...]


_norm_split = pl.pallas_call(
    _norm_split_body,
    grid=(2, NBLK),
    in_specs=[
        pl.BlockSpec((BLK, HALF), lambda i, j: (j, i)),
        pl.BlockSpec((8, HALF), lambda i, j: (0, i)),
        pl.BlockSpec((1, HALF), lambda i, j: (0, i)),
        pl.BlockSpec((1, HALF), lambda i, j: (0, i)),
    ],
    out_specs=pl.BlockSpec((BLK, HALF), lambda i, j: (i * NBLK + j, 0)),
    out_shape=jax.ShapeDtypeStruct((2 * N, HALF), jnp.float32),
)


def _norm_full_body(z_ref, stats_ref, g_ref, be_ref, out_ref):
    mean = stats_ref[0:1, :] / N
    var = stats_ref[1:2, :] / N - mean * mean
    inv = g_ref[...] * lax.rsqrt(var + 1e-5)
    out_ref[...] = (z_ref[...] - mean) * inv + be_ref[...]


_norm_full = pl.pallas_call(
    _norm_full_body,
    grid=(NBLK,),
    in_specs=[
        pl.BlockSpec((BLK, DIM), lambda j: (j, 0)),
        pl.BlockSpec((8, DIM), lambda j: (0, 0)),
        pl.BlockSpec((1, DIM), lambda j: (0, 0)),
        pl.BlockSpec((1, DIM), lambda j: (0, 0)),
    ],
    out_specs=pl.BlockSpec((BLK, DIM), lambda j: (j, 0)),
    out_shape=jax.ShapeDtypeStruct((N, DIM), jnp.float32),
)


def kernel(x, edge_index, w11, b11, w12, b12, g1, be1,
           w21, b21, w22, b22, g2, be2,
           w31, b31, w32, b32, g3, be3):
    src = edge_index[0].astype(jnp.int32)
    dst = edge_index[1].astype(jnp.int32)
    pad = PADE - E
    src = jnp.concatenate([src, jnp.zeros((pad,), jnp.int32)])
    dst = jnp.concatenate([dst, jnp.full((pad,), N, jnp.int32)])
    src2 = jnp.stack([src, src + N]).reshape(2, NCHUNKS, CHUNK)
    dstc = dst.reshape(NCHUNKS, CHUNK)

    h = jnp.concatenate([x[:, :HALF], x[:, HALF:]], axis=0)

    layers = [(w11, b11, w12, b12, g1, be1),
              (w21, b21, w22, b22, g2, be2),
              (w31, b31, w32, b32, g3, be3)]

    zer = jnp.zeros((RPS, HALF), jnp.float32)
    for li, (w1, b1, w2, b2, g, be) in enumerate(layers):
        agg = _sc_aggregate(src2, dstc, h, zer)
        z, stats = _dense(h, h, agg, agg, w1, b1.reshape(1, DIM),
                          w2, b2.reshape(1, DIM))
        if li < 2:
            h = _norm_split(z, stats, g.reshape(1, DIM), be.reshape(1, DIM))
        else:
            return _norm_full(z, stats, g.reshape(1, DIM), be.reshape(1, DIM))
